# Initial kernel scaffold; baseline (speedup 1.0000x reference)
import jax
import jax.numpy as jnp
from jax.experimental import pallas as pl


def kernel(x, hierarchy_mapping, num_fine_nodes):
    raise NotImplementedError("write your pallas kernel here")


# speedup vs baseline: 21.0838x; 21.0838x over previous
import functools

import jax
import jax.numpy as jnp
from jax import lax
from jax.experimental import pallas as pl
from jax.experimental.pallas import tpu as pltpu
from jax.experimental.pallas import tpu_sc as plsc

_B = 2
_C = 10000
_F = 2
_H = 128
_D = _F * _H
_N = 50000
_CHUNK = 128
_NCHUNKS = (_N + _CHUNK - 1) // _CHUNK
_NW = 32
_ITERS = (_NCHUNKS + _NW - 1) // _NW

_mesh = plsc.VectorSubcoreMesh(core_axis_name="c", subcore_axis_name="s")


@functools.partial(
    pl.kernel,
    mesh=_mesh,
    out_type=jax.ShapeDtypeStruct((_B, _N, _D), jnp.float32),
    scratch_types=[
        pltpu.VMEM((_CHUNK,), jnp.int32),
        pltpu.VMEM((_CHUNK, _D), jnp.float32),
        pltpu.SemaphoreType.DMA,
    ],
)
def _unpool(x_hbm, idx_hbm, out_hbm, idx_v, rows_v, sem):
    wid = lax.axis_index("s") * 2 + lax.axis_index("c")

    def body(j, carry):
        chunk = wid + j * _NW

        @pl.when(chunk < _NCHUNKS)
        def _():
            base = jnp.minimum(chunk * _CHUNK, _N - _CHUNK)
            base = pl.multiple_of(base, 8)
            pltpu.sync_copy(idx_hbm.at[pl.ds(base, _CHUNK)], idx_v)
            for b in range(_B):
                pltpu.async_copy(x_hbm.at[b].at[idx_v], rows_v, sem).wait()
                pltpu.sync_copy(rows_v, out_hbm.at[b, pl.ds(base, _CHUNK)])

        return carry

    lax.fori_loop(0, _ITERS, body, 0)


def kernel(x, hierarchy_mapping, num_fine_nodes):
    idx = hierarchy_mapping.astype(jnp.int32)
    x2 = x.reshape(_B, _C, _D)
    out = _unpool(x2, idx)
    return out.reshape(_B, _N, _F, _H)


# speedup vs baseline: 22.9941x; 1.0906x over previous
import functools

import jax
import jax.numpy as jnp
from jax import lax
from jax.experimental import pallas as pl
from jax.experimental.pallas import tpu as pltpu
from jax.experimental.pallas import tpu_sc as plsc

_B = 2
_C = 10000
_F = 2
_H = 128
_D = _F * _H
_N = 50000
_CHUNK = 128
_NCHUNKS = (_N + _CHUNK - 1) // _CHUNK
_NW = 32
_ITERS = (_NCHUNKS + _NW - 1) // _NW
_NFULL = _ITERS - 1
_NTAIL_W = _NCHUNKS - _NFULL * _NW
_NBUF = 3

_mesh = plsc.VectorSubcoreMesh(core_axis_name="c", subcore_axis_name="s")


@functools.partial(
    pl.kernel,
    mesh=_mesh,
    out_type=jax.ShapeDtypeStruct((_B, _N, _D), jnp.float32),
    scratch_types=[
        pltpu.VMEM((_ITERS, _CHUNK), jnp.int32),
        pltpu.VMEM((_CHUNK, _D), jnp.float32),
        pltpu.VMEM((_CHUNK, _D), jnp.float32),
        pltpu.VMEM((_CHUNK, _D), jnp.float32),
        pltpu.SemaphoreType.DMA,
        pltpu.SemaphoreType.DMA,
        pltpu.SemaphoreType.DMA,
        pltpu.SemaphoreType.DMA,
        pltpu.SemaphoreType.DMA,
        pltpu.SemaphoreType.DMA,
        pltpu.SemaphoreType.DMA,
    ],
)
def _unpool(x_hbm, idx_hbm, out_hbm, idx_v, buf0, buf1, buf2,
            isem, gs0, gs1, gs2, ss0, ss1, ss2):
    bufs = (buf0, buf1, buf2)
    gsems = (gs0, gs1, gs2)
    ssems = (ss0, ss1, ss2)
    wid = lax.axis_index("s") * 2 + lax.axis_index("c")

    def base_of(j):
        base = jnp.minimum((wid + j * _NW) * _CHUNK, _N - _CHUNK)
        return pl.multiple_of(base, 8)

    icopies = [
        pltpu.async_copy(idx_hbm.at[pl.ds(base_of(j), _CHUNK)], idx_v.at[j], isem)
        for j in range(_ITERS)
    ]
    for c in icopies:
        c.wait()

    def gather(t, j, b):
        return pltpu.async_copy(
            x_hbm.at[b].at[idx_v.at[j]], bufs[t % _NBUF], gsems[t % _NBUF])

    def scatter(t, j, b):
        return pltpu.async_copy(
            bufs[t % _NBUF], out_hbm.at[b, pl.ds(base_of(j), _CHUNK)],
            ssems[t % _NBUF])

    ntasks = _NFULL * _B
    gd = {}
    sd = {}
    for t in range(ntasks):
        j, b = divmod(t, _B)
        if t >= _NBUF:
            sd[t - _NBUF].wait()
        gd[t] = gather(t, j, b)
        if t >= 1:
            jp, bp = divmod(t - 1, _B)
            gd[t - 1].wait()
            sd[t - 1] = scatter(t - 1, jp, bp)
    last = ntasks - 1
    gd[last].wait()
    sd[last] = scatter(last, *divmod(last, _B))
    sd[last - 2].wait()
    sd[last - 1].wait()

    @pl.when(wid < _NTAIL_W)
    def _():
        for b in range(_B):
            g = pltpu.async_copy(
                x_hbm.at[b].at[idx_v.at[_NFULL]], bufs[b], gsems[b])
            g.wait()
            s = pltpu.async_copy(
                bufs[b], out_hbm.at[b, pl.ds(base_of(_NFULL), _CHUNK)], ssems[b])
            s.wait()

    sd[last].wait()


def kernel(x, hierarchy_mapping, num_fine_nodes):
    idx = hierarchy_mapping.astype(jnp.int32)
    x2 = x.reshape(_B, _C, _D)
    out = _unpool(x2, idx)
    return out.reshape(_B, _N, _F, _H)


# speedup vs baseline: 56.6474x; 2.4636x over previous
import functools

import jax
import jax.numpy as jnp
from jax import lax
from jax.experimental import pallas as pl
from jax.experimental.pallas import tpu as pltpu
from jax.experimental.pallas import tpu_sc as plsc

_B = 2
_C = 10000
_F = 2
_H = 128
_D = _F * _H
_N = 50000
_CHUNK = 128
_NCHUNKS = (_N + _CHUNK - 1) // _CHUNK
_NW = 32
_ITERS = (_NCHUNKS + _NW - 1) // _NW
_NFULL = _ITERS - 1
_NTAIL_W = _NCHUNKS - _NFULL * _NW
_NBUF = 3

_mesh = plsc.VectorSubcoreMesh(core_axis_name="c", subcore_axis_name="s")


@functools.partial(
    pl.kernel,
    mesh=_mesh,
    out_type=jax.ShapeDtypeStruct((_B, _N, _F, _H), jnp.float32),
    scratch_types=[
        pltpu.VMEM((_ITERS, _CHUNK), jnp.int32),
        pltpu.VMEM((_CHUNK, _F, _H), jnp.float32),
        pltpu.VMEM((_CHUNK, _F, _H), jnp.float32),
        pltpu.VMEM((_CHUNK, _F, _H), jnp.float32),
        pltpu.SemaphoreType.DMA,
        pltpu.SemaphoreType.DMA,
        pltpu.SemaphoreType.DMA,
        pltpu.SemaphoreType.DMA,
        pltpu.SemaphoreType.DMA,
        pltpu.SemaphoreType.DMA,
        pltpu.SemaphoreType.DMA,
    ],
)
def _unpool(x_hbm, idx_hbm, out_hbm, idx_v, buf0, buf1, buf2,
            isem, gs0, gs1, gs2, ss0, ss1, ss2):
    bufs = (buf0, buf1, buf2)
    gsems = (gs0, gs1, gs2)
    ssems = (ss0, ss1, ss2)
    wid = lax.axis_index("s") * 2 + lax.axis_index("c")

    def base_of(j):
        base = jnp.minimum((wid + j * _NW) * _CHUNK, _N - _CHUNK)
        return pl.multiple_of(base, 8)

    icopies = [
        pltpu.async_copy(idx_hbm.at[pl.ds(base_of(j), _CHUNK)], idx_v.at[j], isem)
        for j in range(_ITERS)
    ]
    for c in icopies:
        c.wait()

    def gather(t, j, b):
        return pltpu.async_copy(
            x_hbm.at[b].at[idx_v.at[j]], bufs[t % _NBUF], gsems[t % _NBUF])

    def scatter(t, j, b):
        return pltpu.async_copy(
            bufs[t % _NBUF], out_hbm.at[b, pl.ds(base_of(j), _CHUNK)],
            ssems[t % _NBUF])

    ntasks = _NFULL * _B
    gd = {}
    sd = {}
    for t in range(ntasks):
        j, b = divmod(t, _B)
        if t >= _NBUF:
            sd[t - _NBUF].wait()
        gd[t] = gather(t, j, b)
        if t >= 1:
            jp, bp = divmod(t - 1, _B)
            gd[t - 1].wait()
            sd[t - 1] = scatter(t - 1, jp, bp)
    last = ntasks - 1
    gd[last].wait()
    sd[last] = scatter(last, *divmod(last, _B))
    sd[last - 2].wait()
    sd[last - 1].wait()

    @pl.when(wid < _NTAIL_W)
    def _():
        for b in range(_B):
            g = pltpu.async_copy(
                x_hbm.at[b].at[idx_v.at[_NFULL]], bufs[b], gsems[b])
            g.wait()
            s = pltpu.async_copy(
                bufs[b], out_hbm.at[b, pl.ds(base_of(_NFULL), _CHUNK)], ssems[b])
            s.wait()

    sd[last].wait()


def kernel(x, hierarchy_mapping, num_fine_nodes):
    idx = hierarchy_mapping.astype(jnp.int32)
    return _unpool(x, idx)
